# baseline (speedup 1.0000x reference)
import functools

import jax
import jax.numpy as jnp
from jax import lax
from jax.experimental import pallas as pl
from jax.experimental.pallas import tpu as pltpu
from jax.experimental.pallas import tpu_sc as plsc

NC = 2
NS = 16
NW = NC * NS

EPS = 1e-5
RPC = 2
GRPS = (0, 128)
GRPL = (128, 72)
BBT = 128


def _make_gather(B, S, V, piece, npieces):
    bp = B // npieces
    rows_w = bp // NW
    n_chunks = rows_w // RPC
    mesh = plsc.VectorSubcoreMesh(
        core_axis_name="c", subcore_axis_name="s", num_cores=NC, num_subcores=NS
    )

    @functools.partial(
        pl.kernel,
        out_type=jax.ShapeDtypeStruct((bp * S, 128), jnp.float32),
        mesh=mesh,
        scratch_types=[
            pltpu.VMEM((rows_w, S), jnp.int32),
            pltpu.VMEM((2, RPC * S, 128), jnp.float32),
            pltpu.SemaphoreType.DMA,
            pltpu.SemaphoreType.DMA,
        ],
        compiler_params=pltpu.CompilerParams(use_tc_tiling_on_sc=True),
    )
    def k(idx_hbm, table_hbm, out_hbm, idx_v, rows_v, sem_g, sem_o):
        wid = lax.axis_index("s") * NC + lax.axis_index("c")
        brow0 = wid * rows_w

        pltpu.sync_copy(
            idx_hbm.at[pl.ds(piece * bp + brow0, rows_w)], idx_v
        )

        def start_gathers(g, b):
            for r in range(RPC):
                for o, n in zip(GRPS, GRPL):
                    pltpu.async_copy(
                        table_hbm.at[idx_v.at[g * RPC + r, pl.ds(o, n)]],
                        rows_v.at[b, pl.ds(r * S + o, n)],
                        sem_g,
                    )

        def wait_chunk(b, sem):
            pltpu.make_async_copy(
                out_hbm.at[pl.ds(0, RPC * S)], rows_v.at[b], sem
            ).wait()

        start_gathers(0, 0)

        def chunk_body(g, carry):
            b = g % 2
            wait_chunk(b, sem_g)

            @pl.when(g + 1 < n_chunks)
            def _():
                @pl.when(g >= 1)
                def _():
                    wait_chunk(1 - b, sem_o)
                start_gathers(g + 1, 1 - b)

            pltpu.async_copy(
                rows_v.at[b],
                out_hbm.at[pl.ds((brow0 + g * RPC) * S, RPC * S)],
                sem_o,
            )
            return carry

        lax.fori_loop(0, n_chunks, chunk_body, 0)
        wait_chunk(0, sem_o)
        wait_chunk(1, sem_o)

    return k


def _pad_body(t_ref, o_ref):
    o_ref[:, :64] = t_ref[...]
    o_ref[:, 64:] = jnp.zeros_like(t_ref[...])


def _make_pad(V, D):
    BLK = 8000
    assert V % BLK == 0
    return pl.pallas_call(
        _pad_body,
        grid=(V // BLK,),
        in_specs=[pl.BlockSpec((BLK, D), lambda i: (i, 0))],
        out_specs=pl.BlockSpec((BLK, 128), lambda i: (i, 0)),
        out_shape=jax.ShapeDtypeStruct((V, 128), jnp.float32),
        compiler_params=pltpu.CompilerParams(
            dimension_semantics=("arbitrary",),
        ),
    )


def _ln_body(e_ref, gb_ref, out_ref):
    x = e_ref[...][:, :64]
    gam = gb_ref[0, :]
    bet = gb_ref[1, :]
    inv_d = jnp.float32(1.0 / 64)
    s = jnp.sum(x, axis=1) * inv_d
    q = jnp.sum(x * x, axis=1) * inv_d
    r = lax.rsqrt(q - s * s + jnp.float32(EPS))
    nm = (x - s[:, None]) * r[:, None] * gam[None, :] + bet[None, :]
    nrows, S, D = out_ref.shape[2], out_ref.shape[0], out_ref.shape[1]
    out_ref[...] = jnp.transpose(nm.reshape(nrows, S, D), (1, 2, 0))


def _ln_body_acc(e_ref, gb_ref, prev_ref, out_ref):
    _ln_body(e_ref, gb_ref, out_ref)


def _make_ln(B, S, D, piece, npieces, first):
    bp = B // npieces
    steps_p = bp // BBT
    body = _ln_body if first else _ln_body_acc
    in_specs = [
        pl.BlockSpec((BBT * S, 128), lambda i: (i, 0)),
        pl.BlockSpec((2, D), lambda i: (0, 0)),
    ]
    if not first:
        in_specs.append(pl.BlockSpec((8, D, 128), lambda i: (0, 0, 0)))
    return pl.pallas_call(
        body,
        grid=(steps_p,),
        in_specs=in_specs,
        out_specs=pl.BlockSpec(
            (S, D, BBT), lambda i, p=piece: (0, 0, p * steps_p + i)
        ),
        out_shape=jax.ShapeDtypeStruct((S, D, B), jnp.float32),
        input_output_aliases={} if first else {2: 0},
        compiler_params=pltpu.CompilerParams(
            dimension_semantics=("arbitrary",),
            vmem_limit_bytes=100 * 1024 * 1024,
        ),
    )


NP = 8


def kernel(x, table, gamma, beta):
    B, S = x.shape
    V, D = table.shape
    tp = jnp.concatenate([table, jnp.zeros((V, 128 - D), jnp.float32)], axis=1)
    xi = x.astype(jnp.int32)
    gb = jnp.stack([gamma, beta]).astype(jnp.float32)
    es = [_make_gather(B, S, V, p, NP)(xi, tp) for p in range(NP)]
    out_t = _make_ln(B, S, D, 0, NP, True)(es[0], gb)
    for p in range(1, NP):
        out_t = _make_ln(B, S, D, p, NP, False)(es[p], gb, out_t)
    return out_t.transpose(2, 0, 1)


# speedup vs baseline: 1.0106x; 1.0106x over previous
import functools

import jax
import jax.numpy as jnp
from jax import lax
from jax.experimental import pallas as pl
from jax.experimental.pallas import tpu as pltpu
from jax.experimental.pallas import tpu_sc as plsc

NC = 2
NS = 16
NW = NC * NS

EPS = 1e-5
RPC = 2
GRPS = (0, 128)
GRPL = (128, 72)
BBT = 128


def _make_gather(B, S, V, piece, npieces):
    bp = B // npieces
    rows_w = bp // NW
    n_chunks = rows_w // RPC
    mesh = plsc.VectorSubcoreMesh(
        core_axis_name="c", subcore_axis_name="s", num_cores=NC, num_subcores=NS
    )

    @functools.partial(
        pl.kernel,
        out_type=jax.ShapeDtypeStruct((bp * S, 128), jnp.float32),
        mesh=mesh,
        scratch_types=[
            pltpu.VMEM((rows_w, S), jnp.int32),
            pltpu.VMEM((2, RPC * S, 128), jnp.float32),
            pltpu.SemaphoreType.DMA,
            pltpu.SemaphoreType.DMA,
        ],
        compiler_params=pltpu.CompilerParams(use_tc_tiling_on_sc=True),
    )
    def k(idx_hbm, table_hbm, out_hbm, idx_v, rows_v, sem_g, sem_o):
        wid = lax.axis_index("s") * NC + lax.axis_index("c")
        brow0 = wid * rows_w

        pltpu.sync_copy(
            idx_hbm.at[pl.ds(piece * bp + brow0, rows_w)], idx_v
        )

        def start_gathers(g, b):
            for r in range(RPC):
                for o, n in zip(GRPS, GRPL):
                    pltpu.async_copy(
                        table_hbm.at[idx_v.at[g * RPC + r, pl.ds(o, n)]],
                        rows_v.at[b, pl.ds(r * S + o, n)],
                        sem_g,
                    )

        def wait_chunk(b, sem):
            pltpu.make_async_copy(
                out_hbm.at[pl.ds(0, RPC * S)], rows_v.at[b], sem
            ).wait()

        start_gathers(0, 0)

        def chunk_body(g, carry):
            b = g % 2
            wait_chunk(b, sem_g)

            @pl.when(g + 1 < n_chunks)
            def _():
                @pl.when(g >= 1)
                def _():
                    wait_chunk(1 - b, sem_o)
                start_gathers(g + 1, 1 - b)

            pltpu.async_copy(
                rows_v.at[b],
                out_hbm.at[pl.ds((brow0 + g * RPC) * S, RPC * S)],
                sem_o,
            )
            return carry

        lax.fori_loop(0, n_chunks, chunk_body, 0)
        wait_chunk(0, sem_o)
        wait_chunk(1, sem_o)

    return k


def _pad_body(t_ref, o_ref):
    o_ref[:, :64] = t_ref[...]
    o_ref[:, 64:] = jnp.zeros_like(t_ref[...])


def _make_pad(V, D):
    BLK = 8000
    assert V % BLK == 0
    return pl.pallas_call(
        _pad_body,
        grid=(V // BLK,),
        in_specs=[pl.BlockSpec((BLK, D), lambda i: (i, 0))],
        out_specs=pl.BlockSpec((BLK, 128), lambda i: (i, 0)),
        out_shape=jax.ShapeDtypeStruct((V, 128), jnp.float32),
        compiler_params=pltpu.CompilerParams(
            dimension_semantics=("arbitrary",),
        ),
    )


def _ln_body(e_ref, gb_ref, out_ref):
    x = e_ref[...][:, :64]
    gam = gb_ref[0, :]
    bet = gb_ref[1, :]
    inv_d = jnp.float32(1.0 / 64)
    s = jnp.sum(x, axis=1) * inv_d
    q = jnp.sum(x * x, axis=1) * inv_d
    r = lax.rsqrt(q - s * s + jnp.float32(EPS))
    nm = (x - s[:, None]) * r[:, None] * gam[None, :] + bet[None, :]
    nrows, S, D = out_ref.shape[2], out_ref.shape[0], out_ref.shape[1]
    out_ref[...] = jnp.transpose(nm.reshape(nrows, S, D), (1, 2, 0))


def _ln_body_acc(e_ref, gb_ref, prev_ref, out_ref):
    _ln_body(e_ref, gb_ref, out_ref)


def _make_ln(B, S, D, piece, npieces, first):
    bp = B // npieces
    steps_p = bp // BBT
    body = _ln_body if first else _ln_body_acc
    in_specs = [
        pl.BlockSpec((BBT * S, 128), lambda i: (i, 0)),
        pl.BlockSpec((2, D), lambda i: (0, 0)),
    ]
    if not first:
        in_specs.append(pl.BlockSpec((8, D, 128), lambda i: (0, 0, 0)))
    return pl.pallas_call(
        body,
        grid=(steps_p,),
        in_specs=in_specs,
        out_specs=pl.BlockSpec(
            (S, D, BBT), lambda i, p=piece: (0, 0, p * steps_p + i)
        ),
        out_shape=jax.ShapeDtypeStruct((S, D, B), jnp.float32),
        input_output_aliases={} if first else {2: 0},
        compiler_params=pltpu.CompilerParams(
            dimension_semantics=("arbitrary",),
            vmem_limit_bytes=100 * 1024 * 1024,
        ),
    )


NP = 4


def kernel(x, table, gamma, beta):
    B, S = x.shape
    V, D = table.shape
    tp = jnp.concatenate([table, jnp.zeros((V, 128 - D), jnp.float32)], axis=1)
    xi = x.astype(jnp.int32)
    gb = jnp.stack([gamma, beta]).astype(jnp.float32)
    es = [_make_gather(B, S, V, p, NP)(xi, tp) for p in range(NP)]
    out_t = _make_ln(B, S, D, 0, NP, True)(es[0], gb)
    for p in range(1, NP):
        out_t = _make_ln(B, S, D, p, NP, False)(es[p], gb, out_t)
    return out_t.transpose(2, 0, 1)
